# baseline (speedup 1.0000x reference)
import functools

import jax
import jax.numpy as jnp
from jax import lax
from jax.experimental import pallas as pl
from jax.experimental.pallas import tpu as pltpu
from jax.experimental.pallas import tpu_sc as plsc

S = 8
K = 8192
D = 256
N = 8192
M = 2048
KC = 2048
NW = 32
CH = 128
LANES = 16


def _tc_body(n, d, k_total, m, kc, r_ref, e_ref, idx_ref, zsum_ref):
    kblk = k_total // kc
    r = r_ref[...]
    zsq = jnp.sum(r * r, axis=1, keepdims=True)
    zsum_ref[0] = zsq

    bval = jnp.full((m, 1), jnp.inf, jnp.float32)
    bidx = jnp.zeros((m, 1), jnp.int32)
    for kb in range(kblk):
        e = e_ref[kb * kc:(kb + 1) * kc, :]
        mm = lax.dot_general(r, e, (((1,), (1,)), ((), ())),
                             preferred_element_type=jnp.float32)
        esq = jnp.sum(e * e, axis=1)[None, :]
        dist = (zsq + esq) - 2.0 * mm
        lmin = jnp.min(dist, axis=1, keepdims=True)
        lanes = lax.broadcasted_iota(jnp.int32, (m, kc), 1) + kb * kc
        cand = jnp.where(dist == lmin, lanes, jnp.int32(2**31 - 1))
        larg = jnp.min(cand, axis=1, keepdims=True)
        better = lmin < bval
        bval = jnp.where(better, lmin, bval)
        bidx = jnp.where(better, larg, bidx)
    idx_ref[0] = bidx


def _make_tc_stage(n, d, k_total, m, kc, interpret=False):
    nblk = n // m
    return pl.pallas_call(
        functools.partial(_tc_body, n, d, k_total, m, kc),
        grid=(nblk,),
        in_specs=[
            pl.BlockSpec((m, d), lambda ib: (ib, 0)),
            pl.BlockSpec((k_total, d), lambda ib: (0, 0)),
        ],
        out_specs=[
            pl.BlockSpec((1, m, 1), lambda ib: (ib, 0, 0)),
            pl.BlockSpec((1, m, 1), lambda ib: (ib, 0, 0)),
        ],
        out_shape=[
            jax.ShapeDtypeStruct((nblk, m, 1), jnp.int32),
            jax.ShapeDtypeStruct((nblk, m, 1), jnp.float32),
        ],
        interpret=interpret,
    )


_tc_stage = _make_tc_stage(N, D, K, M, KC)


def _sc_mesh():
    return plsc.VectorSubcoreMesh(core_axis_name="c", subcore_axis_name="s")


def _wid(nc):
    return lax.axis_index("s") * nc + lax.axis_index("c")


def _sc_update_body(nc, rpw, e_hbm, idx_hbm, r_hbm, out_hbm,
                    idx_v, rows_v, r_v, sem):
    wid = _wid(nc)
    for c in range(rpw // CH):
        base = wid * rpw + c * CH
        pltpu.sync_copy(idx_hbm.at[pl.ds(base, CH)], idx_v)
        pltpu.async_copy(e_hbm.at[idx_v], rows_v, sem).wait()
        pltpu.sync_copy(r_hbm.at[pl.ds(base, CH)], r_v)

        def body(rr, carry):
            for cc in range(D // LANES):
                sl = pl.ds(cc * LANES, LANES)
                r_v[rr, sl] = r_v[rr, sl] - rows_v[rr, sl]
            return carry

        lax.fori_loop(0, CH, body, 0)
        pltpu.sync_copy(r_v, out_hbm.at[pl.ds(base, CH)])


def _sc_final_body(nc, rpw, e_hbm, idx_hbm, r_hbm, z_hbm, q_hbm, ssq_hbm,
                   idx_v, rows_v, r_v, z_v, ssq_v, sem):
    wid = _wid(nc)
    acc = jnp.zeros((LANES,), jnp.float32)
    for c in range(rpw // CH):
        base = wid * rpw + c * CH
        pltpu.sync_copy(idx_hbm.at[pl.ds(base, CH)], idx_v)
        pltpu.async_copy(e_hbm.at[idx_v], rows_v, sem).wait()
        pltpu.sync_copy(r_hbm.at[pl.ds(base, CH)], r_v)
        pltpu.sync_copy(z_hbm.at[pl.ds(base, CH)], z_v)

        def body(rr, carry):
            for cc in range(D // LANES):
                sl = pl.ds(cc * LANES, LANES)
                rn = r_v[rr, sl] - rows_v[rr, sl]
                carry = carry + rn * rn
                z_v[rr, sl] = z_v[rr, sl] - rn
            return carry

        acc = lax.fori_loop(0, CH, body, acc)
        pltpu.sync_copy(z_v, q_hbm.at[pl.ds(base, CH)])
    ssq_v[...] = acc
    pltpu.sync_copy(ssq_v, ssq_hbm.at[wid])


@functools.cache
def _make_sc_update():
    info = plsc.get_sparse_core_info()
    nc = info.num_cores
    nw = nc * info.num_subcores
    rpw = N // nw
    return pl.kernel(
        functools.partial(_sc_update_body, nc, rpw),
        out_type=jax.ShapeDtypeStruct((N, D), jnp.float32),
        mesh=_sc_mesh(),
        scratch_types=[
            pltpu.VMEM((CH,), jnp.int32),
            pltpu.VMEM((CH, D), jnp.float32),
            pltpu.VMEM((CH, D), jnp.float32),
            pltpu.SemaphoreType.DMA,
        ],
    )


@functools.cache
def _make_sc_final():
    info = plsc.get_sparse_core_info()
    nc = info.num_cores
    nw = nc * info.num_subcores
    rpw = N // nw
    return pl.kernel(
        functools.partial(_sc_final_body, nc, rpw),
        out_type=(jax.ShapeDtypeStruct((N, D), jnp.float32),
                  jax.ShapeDtypeStruct((nw, LANES), jnp.float32)),
        mesh=_sc_mesh(),
        scratch_types=[
            pltpu.VMEM((CH,), jnp.int32),
            pltpu.VMEM((CH, D), jnp.float32),
            pltpu.VMEM((CH, D), jnp.float32),
            pltpu.VMEM((CH, D), jnp.float32),
            pltpu.VMEM((LANES,), jnp.float32),
            pltpu.SemaphoreType.DMA,
        ],
    )


def kernel(z_e, codebooks):
    B, _, T = z_e.shape
    z_flat = jnp.transpose(z_e, (0, 2, 1)).reshape(N, D)
    r = z_flat
    idx_list = []
    zsums = []
    ssq = None
    for s in range(S):
        idx3, zsum = _tc_stage(r, codebooks[s])
        idx = idx3.reshape(N)
        idx_list.append(idx.reshape(B, T))
        if s > 0:
            zsums.append(jnp.sum(zsum))
        if s < S - 1:
            r = _make_sc_update()(codebooks[s], idx, r)
        else:
            q_flat, ssq = _make_sc_final()(codebooks[s], idx, r, z_flat)
    loss = (sum(zsums) + jnp.sum(ssq)) * (0.25 / S / (N * D))
    quantized = jnp.transpose(q_flat.reshape(B, T, D), (0, 2, 1))
    indices = jnp.stack(idx_list, axis=1)
    return quantized, loss, indices


# speedup vs baseline: 1.0776x; 1.0776x over previous
import functools

import jax
import jax.numpy as jnp
from jax import lax
from jax.experimental import pallas as pl
from jax.experimental.pallas import tpu as pltpu
from jax.experimental.pallas import tpu_sc as plsc

S = 8
K = 8192
D = 256
N = 8192
M = 1024
KC = 4096
NW = 32
CH = 128
LANES = 16


def _tc_body(n, d, k_total, m, kc, r_ref, e_ref, idx_ref, zsum_ref):
    kblk = k_total // kc
    r = r_ref[...]
    zsq = jnp.sum(r * r, axis=1, keepdims=True)
    zsum_ref[0] = zsq

    bval = jnp.full((m, 1), jnp.inf, jnp.float32)
    bidx = jnp.zeros((m, 1), jnp.int32)
    for kb in range(kblk):
        e = e_ref[kb * kc:(kb + 1) * kc, :]
        mm = lax.dot_general(r, e, (((1,), (1,)), ((), ())),
                             preferred_element_type=jnp.float32)
        esq = jnp.sum(e * e, axis=1)[None, :]
        dist = (zsq + esq) - 2.0 * mm
        lmin = jnp.min(dist, axis=1, keepdims=True)
        lanes = lax.broadcasted_iota(jnp.int32, (m, kc), 1) + kb * kc
        cand = jnp.where(dist == lmin, lanes, jnp.int32(2**31 - 1))
        larg = jnp.min(cand, axis=1, keepdims=True)
        better = lmin < bval
        bval = jnp.where(better, lmin, bval)
        bidx = jnp.where(better, larg, bidx)
    idx_ref[0] = bidx


def _make_tc_stage(n, d, k_total, m, kc, interpret=False):
    nblk = n // m
    return pl.pallas_call(
        functools.partial(_tc_body, n, d, k_total, m, kc),
        grid=(nblk,),
        in_specs=[
            pl.BlockSpec((m, d), lambda ib: (ib, 0)),
            pl.BlockSpec((k_total, d), lambda ib: (0, 0)),
        ],
        out_specs=[
            pl.BlockSpec((1, m, 1), lambda ib: (ib, 0, 0)),
            pl.BlockSpec((1, m, 1), lambda ib: (ib, 0, 0)),
        ],
        out_shape=[
            jax.ShapeDtypeStruct((nblk, m, 1), jnp.int32),
            jax.ShapeDtypeStruct((nblk, m, 1), jnp.float32),
        ],
        interpret=interpret,
    )


_tc_stage = _make_tc_stage(N, D, K, M, KC)


def _sc_mesh():
    return plsc.VectorSubcoreMesh(core_axis_name="c", subcore_axis_name="s")


def _wid(nc):
    return lax.axis_index("s") * nc + lax.axis_index("c")


def _sc_update_body(nc, rpw, e_hbm, idx_hbm, r_hbm, out_hbm,
                    idx_v, rows_v, r_v, sem):
    wid = _wid(nc)
    for c in range(rpw // CH):
        base = wid * rpw + c * CH
        pltpu.sync_copy(idx_hbm.at[pl.ds(base, CH)], idx_v)
        pltpu.async_copy(e_hbm.at[idx_v], rows_v, sem).wait()
        pltpu.sync_copy(r_hbm.at[pl.ds(base, CH)], r_v)

        def body(rr, carry):
            for cc in range(D // LANES):
                sl = pl.ds(cc * LANES, LANES)
                r_v[rr, sl] = r_v[rr, sl] - rows_v[rr, sl]
            return carry

        lax.fori_loop(0, CH, body, 0)
        pltpu.sync_copy(r_v, out_hbm.at[pl.ds(base, CH)])


def _sc_final_body(nc, rpw, e_hbm, idx_hbm, r_hbm, z_hbm, q_hbm, ssq_hbm,
                   idx_v, rows_v, r_v, z_v, ssq_v, sem):
    wid = _wid(nc)
    acc = jnp.zeros((LANES,), jnp.float32)
    for c in range(rpw // CH):
        base = wid * rpw + c * CH
        pltpu.sync_copy(idx_hbm.at[pl.ds(base, CH)], idx_v)
        pltpu.async_copy(e_hbm.at[idx_v], rows_v, sem).wait()
        pltpu.sync_copy(r_hbm.at[pl.ds(base, CH)], r_v)
        pltpu.sync_copy(z_hbm.at[pl.ds(base, CH)], z_v)

        def body(rr, carry):
            for cc in range(D // LANES):
                sl = pl.ds(cc * LANES, LANES)
                rn = r_v[rr, sl] - rows_v[rr, sl]
                carry = carry + rn * rn
                z_v[rr, sl] = z_v[rr, sl] - rn
            return carry

        acc = lax.fori_loop(0, CH, body, acc)
        pltpu.sync_copy(z_v, q_hbm.at[pl.ds(base, CH)])
    ssq_v[...] = acc
    pltpu.sync_copy(ssq_v, ssq_hbm.at[wid])


@functools.cache
def _make_sc_update():
    info = plsc.get_sparse_core_info()
    nc = info.num_cores
    nw = nc * info.num_subcores
    rpw = N // nw
    return pl.kernel(
        functools.partial(_sc_update_body, nc, rpw),
        out_type=jax.ShapeDtypeStruct((N, D), jnp.float32),
        mesh=_sc_mesh(),
        scratch_types=[
            pltpu.VMEM((CH,), jnp.int32),
            pltpu.VMEM((CH, D), jnp.float32),
            pltpu.VMEM((CH, D), jnp.float32),
            pltpu.SemaphoreType.DMA,
        ],
    )


@functools.cache
def _make_sc_final():
    info = plsc.get_sparse_core_info()
    nc = info.num_cores
    nw = nc * info.num_subcores
    rpw = N // nw
    return pl.kernel(
        functools.partial(_sc_final_body, nc, rpw),
        out_type=(jax.ShapeDtypeStruct((N, D), jnp.float32),
                  jax.ShapeDtypeStruct((nw, LANES), jnp.float32)),
        mesh=_sc_mesh(),
        scratch_types=[
            pltpu.VMEM((CH,), jnp.int32),
            pltpu.VMEM((CH, D), jnp.float32),
            pltpu.VMEM((CH, D), jnp.float32),
            pltpu.VMEM((CH, D), jnp.float32),
            pltpu.VMEM((LANES,), jnp.float32),
            pltpu.SemaphoreType.DMA,
        ],
    )


def kernel(z_e, codebooks):
    B, _, T = z_e.shape
    z_flat = jnp.transpose(z_e, (0, 2, 1)).reshape(N, D)
    r = z_flat
    idx_list = []
    zsums = []
    ssq = None
    for s in range(S):
        idx3, zsum = _tc_stage(r, codebooks[s])
        idx = idx3.reshape(N)
        idx_list.append(idx.reshape(B, T))
        if s > 0:
            zsums.append(jnp.sum(zsum))
        if s < S - 1:
            r = _make_sc_update()(codebooks[s], idx, r)
        else:
            q_flat, ssq = _make_sc_final()(codebooks[s], idx, r, z_flat)
    loss = (sum(zsums) + jnp.sum(ssq)) * (0.25 / S / (N * D))
    quantized = jnp.transpose(q_flat.reshape(B, T, D), (0, 2, 1))
    indices = jnp.stack(idx_list, axis=1)
    return quantized, loss, indices


# speedup vs baseline: 1.1087x; 1.0289x over previous
import functools

import jax
import jax.numpy as jnp
from jax import lax
from jax.experimental import pallas as pl
from jax.experimental.pallas import tpu as pltpu
from jax.experimental.pallas import tpu_sc as plsc

S = 8
K = 8192
D = 256
N = 8192
M = 1024
KC = 8192
NW = 32
CH = 128
LANES = 16


def _tc_body(n, d, k_total, m, kc, r_ref, e_ref, idx_ref, zsum_ref):
    kblk = k_total // kc
    r = r_ref[...]
    zsq = jnp.sum(r * r, axis=1, keepdims=True)
    zsum_ref[0] = zsq

    bval = jnp.full((m, 1), jnp.inf, jnp.float32)
    bidx = jnp.zeros((m, 1), jnp.int32)
    for kb in range(kblk):
        e = e_ref[kb * kc:(kb + 1) * kc, :]
        mm = lax.dot_general(r, e, (((1,), (1,)), ((), ())),
                             preferred_element_type=jnp.float32)
        esq = jnp.sum(e * e, axis=1)[None, :]
        dist = (zsq + esq) - 2.0 * mm
        lmin = jnp.min(dist, axis=1, keepdims=True)
        lanes = lax.broadcasted_iota(jnp.int32, (m, kc), 1) + kb * kc
        cand = jnp.where(dist == lmin, lanes, jnp.int32(2**31 - 1))
        larg = jnp.min(cand, axis=1, keepdims=True)
        better = lmin < bval
        bval = jnp.where(better, lmin, bval)
        bidx = jnp.where(better, larg, bidx)
    idx_ref[0] = bidx


def _make_tc_stage(n, d, k_total, m, kc, interpret=False):
    nblk = n // m
    return pl.pallas_call(
        functools.partial(_tc_body, n, d, k_total, m, kc),
        grid=(nblk,),
        in_specs=[
            pl.BlockSpec((m, d), lambda ib: (ib, 0)),
            pl.BlockSpec((k_total, d), lambda ib: (0, 0)),
        ],
        out_specs=[
            pl.BlockSpec((1, m, 1), lambda ib: (ib, 0, 0)),
            pl.BlockSpec((1, m, 1), lambda ib: (ib, 0, 0)),
        ],
        out_shape=[
            jax.ShapeDtypeStruct((nblk, m, 1), jnp.int32),
            jax.ShapeDtypeStruct((nblk, m, 1), jnp.float32),
        ],
        interpret=interpret,
    )


_tc_stage = _make_tc_stage(N, D, K, M, KC)


def _sc_mesh():
    return plsc.VectorSubcoreMesh(core_axis_name="c", subcore_axis_name="s")


def _wid(nc):
    return lax.axis_index("s") * nc + lax.axis_index("c")


def _sc_update_body(nc, rpw, e_hbm, idx_hbm, r_hbm, out_hbm,
                    idx_v, rows_v, r_v, sem):
    wid = _wid(nc)
    for c in range(rpw // CH):
        base = wid * rpw + c * CH
        pltpu.sync_copy(idx_hbm.at[pl.ds(base, CH)], idx_v)
        pltpu.async_copy(e_hbm.at[idx_v], rows_v, sem).wait()
        pltpu.sync_copy(r_hbm.at[pl.ds(base, CH)], r_v)

        def body(rr, carry):
            for cc in range(D // LANES):
                sl = pl.ds(cc * LANES, LANES)
                r_v[rr, sl] = r_v[rr, sl] - rows_v[rr, sl]
            return carry

        lax.fori_loop(0, CH, body, 0)
        pltpu.sync_copy(r_v, out_hbm.at[pl.ds(base, CH)])


def _sc_final_body(nc, rpw, e_hbm, idx_hbm, r_hbm, z_hbm, q_hbm, ssq_hbm,
                   idx_v, rows_v, r_v, z_v, ssq_v, sem):
    wid = _wid(nc)
    acc = jnp.zeros((LANES,), jnp.float32)
    for c in range(rpw // CH):
        base = wid * rpw + c * CH
        pltpu.sync_copy(idx_hbm.at[pl.ds(base, CH)], idx_v)
        pltpu.async_copy(e_hbm.at[idx_v], rows_v, sem).wait()
        pltpu.sync_copy(r_hbm.at[pl.ds(base, CH)], r_v)
        pltpu.sync_copy(z_hbm.at[pl.ds(base, CH)], z_v)

        def body(rr, carry):
            for cc in range(D // LANES):
                sl = pl.ds(cc * LANES, LANES)
                rn = r_v[rr, sl] - rows_v[rr, sl]
                carry = carry + rn * rn
                z_v[rr, sl] = z_v[rr, sl] - rn
            return carry

        acc = lax.fori_loop(0, CH, body, acc)
        pltpu.sync_copy(z_v, q_hbm.at[pl.ds(base, CH)])
    ssq_v[...] = acc
    pltpu.sync_copy(ssq_v, ssq_hbm.at[wid])


@functools.cache
def _make_sc_update():
    info = plsc.get_sparse_core_info()
    nc = info.num_cores
    nw = nc * info.num_subcores
    rpw = N // nw
    return pl.kernel(
        functools.partial(_sc_update_body, nc, rpw),
        out_type=jax.ShapeDtypeStruct((N, D), jnp.float32),
        mesh=_sc_mesh(),
        scratch_types=[
            pltpu.VMEM((CH,), jnp.int32),
            pltpu.VMEM((CH, D), jnp.float32),
            pltpu.VMEM((CH, D), jnp.float32),
            pltpu.SemaphoreType.DMA,
        ],
    )


@functools.cache
def _make_sc_final():
    info = plsc.get_sparse_core_info()
    nc = info.num_cores
    nw = nc * info.num_subcores
    rpw = N // nw
    return pl.kernel(
        functools.partial(_sc_final_body, nc, rpw),
        out_type=(jax.ShapeDtypeStruct((N, D), jnp.float32),
                  jax.ShapeDtypeStruct((nw, LANES), jnp.float32)),
        mesh=_sc_mesh(),
        scratch_types=[
            pltpu.VMEM((CH,), jnp.int32),
            pltpu.VMEM((CH, D), jnp.float32),
            pltpu.VMEM((CH, D), jnp.float32),
            pltpu.VMEM((CH, D), jnp.float32),
            pltpu.VMEM((LANES,), jnp.float32),
            pltpu.SemaphoreType.DMA,
        ],
    )


def kernel(z_e, codebooks):
    B, _, T = z_e.shape
    z_flat = jnp.transpose(z_e, (0, 2, 1)).reshape(N, D)
    r = z_flat
    idx_list = []
    zsums = []
    ssq = None
    for s in range(S):
        idx3, zsum = _tc_stage(r, codebooks[s])
        idx = idx3.reshape(N)
        idx_list.append(idx.reshape(B, T))
        if s > 0:
            zsums.append(jnp.sum(zsum))
        if s < S - 1:
            r = _make_sc_update()(codebooks[s], idx, r)
        else:
            q_flat, ssq = _make_sc_final()(codebooks[s], idx, r, z_flat)
    loss = (sum(zsums) + jnp.sum(ssq)) * (0.25 / S / (N * D))
    quantized = jnp.transpose(q_flat.reshape(B, T, D), (0, 2, 1))
    indices = jnp.stack(idx_list, axis=1)
    return quantized, loss, indices
